# Initial kernel scaffold; baseline (speedup 1.0000x reference)
import jax
import jax.numpy as jnp
from jax.experimental import pallas as pl


def kernel(x, W_tgt, W_pos):
    raise NotImplementedError("write your pallas kernel here")


# speedup vs baseline: 2.3678x; 2.3678x over previous
import functools

import jax
import jax.numpy as jnp
from jax import lax
from jax.experimental import pallas as pl
from jax.experimental.pallas import tpu as pltpu
from jax.experimental.pallas import tpu_sc as plsc

N_VOCAB = 1000000
D_MODEL = 64
SEQLEN = 200
BATCH = 4096

NUM_WORKERS = 32
ROWS_PER_W = (BATCH * SEQLEN) // NUM_WORKERS
CHUNK = SEQLEN
NCHUNK = ROWS_PER_W // CHUNK
LANES = 16


def _make_body():
    mesh = plsc.VectorSubcoreMesh(core_axis_name="c", subcore_axis_name="s")

    @functools.partial(
        pl.kernel,
        mesh=mesh,
        compiler_params=pltpu.CompilerParams(use_tc_tiling_on_sc=False),
        out_type=jax.ShapeDtypeStruct((BATCH * SEQLEN, D_MODEL), jnp.float32),
        scratch_types=[
            pltpu.VMEM((CHUNK,), jnp.int32),
            pltpu.VMEM((CHUNK, D_MODEL), jnp.float32),
            pltpu.VMEM((SEQLEN, D_MODEL), jnp.float32),
            pltpu.SemaphoreType.DMA,
        ],
    )
    def body(xf_hbm, wt_hbm, wp_hbm, out_hbm, idx_v, rows_v, pos_v, sem):
        wid = lax.axis_index("s") * 2 + lax.axis_index("c")
        base = wid * ROWS_PER_W
        pltpu.sync_copy(wp_hbm, pos_v)

        def chunk_body(c, carry):
            off = base + c * CHUNK
            pltpu.sync_copy(xf_hbm.at[pl.ds(off, CHUNK)], idx_v)
            cp1 = pltpu.async_copy(
                wt_hbm.at[idx_v.at[pl.ds(0, 128)]], rows_v.at[pl.ds(0, 128)], sem)
            cp2 = pltpu.async_copy(
                wt_hbm.at[idx_v.at[pl.ds(128, CHUNK - 128)]],
                rows_v.at[pl.ds(128, CHUNK - 128)], sem)
            cp1.wait()
            cp2.wait()

            def add_row(r, carry2):
                for k in range(D_MODEL // LANES):
                    v = pos_v[r, pl.ds(k * LANES, LANES)]
                    plsc.addupdate(rows_v.at[r, pl.ds(k * LANES, LANES)], v)
                return carry2

            lax.fori_loop(0, CHUNK, add_row, 0, unroll=2)
            pltpu.sync_copy(rows_v, out_hbm.at[pl.ds(off, CHUNK)])
            return carry

        lax.fori_loop(0, NCHUNK, chunk_body, 0)

    return body


_body = _make_body()


def kernel(x, W_tgt, W_pos):
    xf = x.reshape(-1).astype(jnp.int32)
    out = _body(xf, W_tgt, W_pos)
    return out.reshape(BATCH, SEQLEN, D_MODEL)


# speedup vs baseline: 2.4236x; 1.0236x over previous
import functools

import jax
import jax.numpy as jnp
from jax import lax
from jax.experimental import pallas as pl
from jax.experimental.pallas import tpu as pltpu
from jax.experimental.pallas import tpu_sc as plsc

N_VOCAB = 1000000
D_MODEL = 64
SEQLEN = 200
BATCH = 4096

NUM_WORKERS = 32
ROWS_PER_W = (BATCH * SEQLEN) // NUM_WORKERS
CHUNK = SEQLEN
NCHUNK = ROWS_PER_W // CHUNK
LANES = 16


def _make_body():
    mesh = plsc.VectorSubcoreMesh(core_axis_name="c", subcore_axis_name="s")

    @functools.partial(
        pl.kernel,
        mesh=mesh,
        compiler_params=pltpu.CompilerParams(use_tc_tiling_on_sc=False),
        out_type=jax.ShapeDtypeStruct((BATCH * SEQLEN, D_MODEL), jnp.float32),
        scratch_types=[
            pltpu.VMEM((CHUNK,), jnp.int32),
            pltpu.VMEM((CHUNK, D_MODEL), jnp.float32),
            pltpu.VMEM_SHARED((SEQLEN, D_MODEL), jnp.float32),
            pltpu.SemaphoreType.DMA,
        ],
    )
    def body(xf_hbm, wt_hbm, wp_hbm, out_hbm, idx_v, rows_v, pos_sh, sem):
        sid = lax.axis_index("s")
        wid = sid * 2 + lax.axis_index("c")
        base = wid * ROWS_PER_W

        @pl.when(sid == 0)
        def _():
            pltpu.sync_copy(wp_hbm, pos_sh)

        plsc.subcore_barrier()

        def chunk_body(c, carry):
            off = base + c * CHUNK
            pltpu.sync_copy(xf_hbm.at[pl.ds(off, CHUNK)], idx_v)
            pltpu.sync_copy(pos_sh, rows_v)
            cp1 = pltpu.async_copy(
                wt_hbm.at[idx_v.at[pl.ds(0, 128)]], rows_v.at[pl.ds(0, 128)],
                sem, add=True)
            cp2 = pltpu.async_copy(
                wt_hbm.at[idx_v.at[pl.ds(128, CHUNK - 128)]],
                rows_v.at[pl.ds(128, CHUNK - 128)], sem, add=True)
            cp1.wait()
            cp2.wait()
            pltpu.sync_copy(rows_v, out_hbm.at[pl.ds(off, CHUNK)])
            return carry

        lax.fori_loop(0, NCHUNK, chunk_body, 0)

    return body


_body = _make_body()


def kernel(x, W_tgt, W_pos):
    xf = x.reshape(-1).astype(jnp.int32)
    out = _body(xf, W_tgt, W_pos)
    return out.reshape(BATCH, SEQLEN, D_MODEL)


# speedup vs baseline: 2.8625x; 1.1811x over previous
import functools

import jax
import jax.numpy as jnp
from jax import lax
from jax.experimental import pallas as pl
from jax.experimental.pallas import tpu as pltpu
from jax.experimental.pallas import tpu_sc as plsc

N_VOCAB = 1000000
D_MODEL = 64
SEQLEN = 200
BATCH = 4096

NUM_WORKERS = 32
ROWS_PER_W = (BATCH * SEQLEN) // NUM_WORKERS
CHUNK = 2 * SEQLEN
NCHUNK = ROWS_PER_W // CHUNK
NB = 4
SUBS = ((0, 128), (128, 128), (256, 128), (384, 16))


def _make_body():
    mesh = plsc.VectorSubcoreMesh(core_axis_name="c", subcore_axis_name="s")

    @functools.partial(
        pl.kernel,
        mesh=mesh,
        compiler_params=pltpu.CompilerParams(use_tc_tiling_on_sc=False),
        out_type=jax.ShapeDtypeStruct((BATCH * SEQLEN, D_MODEL), jnp.float32),
        scratch_types=[
            pltpu.VMEM((NB, CHUNK), jnp.int32),
            pltpu.VMEM((NB, CHUNK, D_MODEL), jnp.float32),
            pltpu.VMEM_SHARED((CHUNK, D_MODEL), jnp.float32),
            pltpu.SemaphoreType.DMA((NB,)),
            pltpu.SemaphoreType.DMA((NB,)),
            pltpu.SemaphoreType.DMA((NB,)),
            pltpu.SemaphoreType.DMA((NB,)),
        ],
    )
    def body(xf_hbm, wt_hbm, wp_hbm, out_hbm, idx_v, rows_v, pos_sh,
             semg, sems, semp, semi):
        sid = lax.axis_index("s")
        wid = sid * 2 + lax.axis_index("c")
        base = wid * ROWS_PER_W

        @pl.when(sid == 0)
        def _():
            pltpu.sync_copy(wp_hbm, pos_sh.at[pl.ds(0, SEQLEN)])
            pltpu.sync_copy(wp_hbm, pos_sh.at[pl.ds(SEQLEN, SEQLEN)])

        plsc.subcore_barrier()

        def issue_gathers(c, b):
            for (o, n) in SUBS:
                pltpu.async_copy(
                    wt_hbm.at[idx_v.at[b, pl.ds(o, n)]],
                    rows_v.at[b, pl.ds(o, n)], semg.at[b], add=True)

        def wait_gathers(b):
            for (o, n) in SUBS:
                pltpu.make_async_copy(
                    wt_hbm.at[idx_v.at[b, pl.ds(o, n)]],
                    rows_v.at[b, pl.ds(o, n)], semg.at[b]).wait()

        pltpu.sync_copy(pos_sh, rows_v.at[0])
        pltpu.sync_copy(pos_sh, rows_v.at[1])
        pltpu.async_copy(xf_hbm.at[pl.ds(base, CHUNK)], idx_v.at[0], semi.at[0])

        def step(i, carry):
            for b in range(NB):
                c = i * NB + b
                bp = (b - 1) % NB
                br = (b + 2) % NB

                @pl.when(c >= 2)
                def _():
                    pltpu.make_async_copy(
                        pos_sh, rows_v.at[b], semp.at[b]).wait()
                pltpu.make_async_copy(
                    xf_hbm.at[pl.ds(base + c * CHUNK, CHUNK)],
                    idx_v.at[b], semi.at[b]).wait()
                issue_gathers(c, b)

                @pl.when(c < NCHUNK - 1)
                def _():
                    pltpu.async_copy(
                        xf_hbm.at[pl.ds(base + (c + 1) * CHUNK, CHUNK)],
                        idx_v.at[(b + 1) % NB], semi.at[(b + 1) % NB])

                @pl.when(c >= 1)
                def _():
                    wait_gathers(bp)
                    pltpu.async_copy(
                        rows_v.at[bp],
                        out_hbm.at[pl.ds(base + (c - 1) * CHUNK, CHUNK)],
                        sems.at[bp])

                @pl.when(c >= 2)
                def _():
                    pltpu.make_async_copy(
                        rows_v.at[br],
                        out_hbm.at[pl.ds(base + (c - 2) * CHUNK, CHUNK)],
                        sems.at[br]).wait()

                @pl.when(c < NCHUNK - 2)
                def _():
                    pltpu.async_copy(pos_sh, rows_v.at[br], semp.at[br])
            return carry

        lax.fori_loop(0, NCHUNK // NB, step, 0)

        bl = (NCHUNK - 1) % NB
        wait_gathers(bl)
        pltpu.sync_copy(
            rows_v.at[bl],
            out_hbm.at[pl.ds(base + (NCHUNK - 1) * CHUNK, CHUNK)])
        blp = (NCHUNK - 2) % NB
        pltpu.make_async_copy(
            rows_v.at[blp],
            out_hbm.at[pl.ds(base + (NCHUNK - 2) * CHUNK, CHUNK)],
            sems.at[blp]).wait()

    return body


_body = _make_body()


def kernel(x, W_tgt, W_pos):
    xf = x.reshape(-1).astype(jnp.int32)
    out = _body(xf, W_tgt, W_pos)
    return out.reshape(BATCH, SEQLEN, D_MODEL)
